# baseline (speedup 1.0000x reference)
import jax
import jax.numpy as jnp
from jax import lax
from jax.experimental import pallas as pl
from jax.experimental.pallas import tpu as pltpu
from jax.experimental.pallas import tpu_sc as plsc

B = 4
N = 4096
K = 3
NK = N * K
D_PC = 256
D_IMG = 256
C_FEAT = D_PC + D_IMG + 1
C_PAD = 520
HIDDEN = 1024
S = 1024
L = 16
NVP = 128
NC = 2
NCHUNK = NK // L
CPR = N // L


def _select_body(vm_hbm, out_hbm, nv_hbm, mbuf, obuf, sbuf, nvbuf):
    wid = lax.axis_index("s") * NC + lax.axis_index("c")

    @pl.when(wid < B)
    def _():
        b = wid
        pltpu.sync_copy(vm_hbm.at[b], mbuf)
        lanes = lax.iota(jnp.int32, L)

        def body(i, carry):
            c1, c0 = carry
            kr = i // CPR
            col = (i - kr * CPR) * L
            v = mbuf[kr, pl.ds(col, L)]
            m = v != 0.0
            mi = m.astype(jnp.int32)
            cs = plsc.cumsum(mi)
            ii = lanes + 1
            pos_valid = c1 + cs - 1
            pos_invalid = (NK - 1) - (c0 + (ii - cs) - 1)
            pos = jnp.where(m, pos_valid, pos_invalid)
            plsc.store_scatter(obuf, [pos], i * L + lanes)
            nv = jnp.sum(mi)
            return c1 + nv, c0 + (L - nv)

        c1, _ = lax.fori_loop(0, NCHUNK, body, (jnp.int32(0), jnp.int32(0)))

        def fin(j, carry):
            p = j * L + lanes
            in_valid = p < c1
            src = jnp.where(in_valid, p, (NK - 1) - (p - c1))
            sbuf[pl.ds(j * L, L)] = plsc.load_gather(obuf, [src])
            return carry

        lax.fori_loop(0, S // L, fin, 0)
        pltpu.sync_copy(sbuf, out_hbm.at[pl.ds(b * S, S)])
        nvbuf[pl.ds(0, L)] = jnp.full((L,), 0, jnp.int32) + c1
        pltpu.sync_copy(nvbuf, nv_hbm.at[pl.ds(b * NVP, NVP)])


_select = pl.kernel(
    _select_body,
    out_type=(
        jax.ShapeDtypeStruct((B * S,), jnp.int32),
        jax.ShapeDtypeStruct((B * NVP,), jnp.int32),
    ),
    mesh=plsc.VectorSubcoreMesh(core_axis_name="c", subcore_axis_name="s"),
    compiler_params=pltpu.CompilerParams(
        needs_layout_passes=False, use_tc_tiling_on_sc=True),
    scratch_types=[
        pltpu.VMEM((K, N), jnp.float32),
        pltpu.VMEM((NK,), jnp.int32),
        pltpu.VMEM((S,), jnp.int32),
        pltpu.VMEM((NVP,), jnp.int32),
    ],
)


def _gather_body(sample, nv_in, pc, img, sxyz, sinds,
                 feat_out, xyz_out, inds_out,
                 n_buf, k_buf, stage2, out_buf, ibuf, xbuf, xout, nvbuf,
                 sem_a, sem_b, sem_o):
    wid = lax.axis_index("s") * NC + lax.axis_index("c")
    b = wid >> 3
    t = wid & 7
    lanes = lax.iota(jnp.int32, L)

    pltpu.sync_copy(sample.at[pl.ds(b * S, S)], n_buf)

    def split(j, carry):
        s = n_buf[pl.ds(j * L, L)]
        n_buf[pl.ds(j * L, L)] = s & (N - 1)
        k_buf[pl.ds(j * L, L)] = s >> 12
        return carry

    lax.fori_loop(0, S // L, split, 0)

    c_base = t * 32

    blocks = [("pc", i) for i in range(8)] + [("img", i) for i in range(8)]
    sems = (sem_a, sem_b)

    def start_stage(idx):
        kind, cb = blocks[idx]
        sl = idx % 2
        c0 = c_base + 4 * cb
        if kind == "pc":
            return [pltpu.async_copy(pc.at[b, pl.ds(c0, 4), :],
                                     stage2.at[sl, 0], sems[sl])]
        return [pltpu.async_copy(img.at[b, k, pl.ds(c0, 4), :],
                                 stage2.at[sl, k], sems[sl])
                for k in range(K)]

    pending = start_stage(0)
    out_pending = []
    for idx in range(16):
        sl = idx % 2
        nxt = start_stage(idx + 1) if idx + 1 < 16 else []
        for d in pending:
            d.wait()
        pending = nxt
        if sl == 0:
            for d in out_pending:
                d.wait()
            out_pending = []
        kind, cb = blocks[idx]
        if kind == "pc":
            def g(j, carry, sl=sl):
                nv = n_buf[pl.ds(j * L, L)]
                for c in range(4):
                    out_buf[sl * 4 + c, pl.ds(j * L, L)] = plsc.load_gather(
                        stage2.at[sl, 0], [jnp.full((L,), c, jnp.int32), nv])
                return carry
        else:
            def g(j, carry, sl=sl):
                nv = n_buf[pl.ds(j * L, L)]
                kv = k_buf[pl.ds(j * L, L)]
                for c in range(4):
                    out_buf[sl * 4 + c, pl.ds(j * L, L)] = plsc.load_gather(
                        stage2.at[sl], [kv, jnp.full((L,), c, jnp.int32), nv])
                return carry
        lax.fori_loop(0, S // L, g, 0)
        if sl == 1:
            kind0, cb0 = blocks[idx - 1]
            row0 = c_base + 4 * cb0 + (0 if kind0 == "pc" else D_PC)
            out_pending = [pltpu.async_copy(
                out_buf, feat_out.at[b, pl.ds(row0, 8), :], sem_o)]
    for d in out_pending:
        d.wait()

    @pl.when(t == 0)
    def _mask_row():
        pltpu.sync_copy(nv_in.at[pl.ds(b * NVP, NVP)], nvbuf)
        nvv = nvbuf[pl.ds(0, L)]

        def gm(j, carry):
            p = j * L + lanes
            out_buf[0, pl.ds(j * L, L)] = (p < nvv).astype(jnp.float32)
            return carry

        lax.fori_loop(0, S // L, gm, 0)
        pltpu.sync_copy(out_buf.at[pl.ds(0, 1), :],
                        feat_out.at[b, pl.ds(C_FEAT - 1, 1), :])

    @pl.when(t == 1)
    def _inds_row():
        pltpu.sync_copy(sinds.at[pl.ds(b * N, N)], ibuf)

        def gi(j, carry):
            nv = n_buf[pl.ds(j * L, L)]
            k_buf[pl.ds(j * L, L)] = plsc.load_gather(ibuf, [nv])
            return carry

        lax.fori_loop(0, S // L, gi, 0)
        pltpu.sync_copy(k_buf, inds_out.at[pl.ds(b * S, S)])

    @pl.when(t == 2)
    def _xyz_rows():
        pltpu.sync_copy(sxyz.at[pl.ds(b * N * 3, N * 3)], xbuf)

        def gx(j, carry):
            p = j * L + lanes
            jv = p // 3
            dv = p - jv * 3
            nv = plsc.load_gather(n_buf, [jv])
            xout[pl.ds(j * L, L)] = plsc.load_gather(xbuf, [nv * 3 + dv])
            return carry

        lax.fori_loop(0, (S * 3) // L, gx, 0)
        pltpu.sync_copy(xout, xyz_out.at[pl.ds(b * S * 3, S * 3)])


_gather = pl.kernel(
    _gather_body,
    out_type=(
        jax.ShapeDtypeStruct((B, C_FEAT, S), jnp.float32),
        jax.ShapeDtypeStruct((B * S * 3,), jnp.float32),
        jax.ShapeDtypeStruct((B * S,), jnp.int32),
    ),
    mesh=plsc.VectorSubcoreMesh(core_axis_name="c", subcore_axis_name="s"),
    compiler_params=pltpu.CompilerParams(
        needs_layout_passes=False, use_tc_tiling_on_sc=True),
    scratch_types=[
        pltpu.VMEM((S,), jnp.int32),
        pltpu.VMEM((S,), jnp.int32),
        pltpu.VMEM((2, K, 4, N), jnp.float32),
        pltpu.VMEM((8, S), jnp.float32),
        pltpu.VMEM((N,), jnp.int32),
        pltpu.VMEM((N * 3,), jnp.float32),
        pltpu.VMEM((S * 3,), jnp.float32),
        pltpu.VMEM((NVP,), jnp.int32),
        pltpu.SemaphoreType.DMA,
        pltpu.SemaphoreType.DMA,
        pltpu.SemaphoreType.DMA,
    ],
)


def _mlp_body(feat_ref, w1_ref, b1_ref, w2_ref, b2_ref,
              img_out_ref, pc_ref, joint_ref):
    x = feat_ref[0, D_PC:C_FEAT, :]
    h = jnp.dot(w1_ref[...].astype(jnp.bfloat16), x.astype(jnp.bfloat16),
                preferred_element_type=jnp.float32)
    h = jnp.maximum(h + b1_ref[...], 0.0)
    o = jnp.dot(w2_ref[...].astype(jnp.bfloat16), h.astype(jnp.bfloat16),
                preferred_element_type=jnp.float32)
    o = jnp.maximum(o + b2_ref[...], 0.0)
    img_out_ref[0] = o
    pcv = feat_ref[0, :D_PC, :]
    pc_ref[0] = pcv
    joint_ref[0, :D_PC, :] = pcv
    joint_ref[0, D_PC:, :] = o


def _mlp(feat, W1, b1c, W2, b2c):
    return pl.pallas_call(
        _mlp_body,
        grid=(B,),
        in_specs=[
            pl.BlockSpec((1, C_FEAT, S), lambda b: (b, 0, 0)),
            pl.BlockSpec((HIDDEN, D_IMG + 1), lambda b: (0, 0)),
            pl.BlockSpec((HIDDEN, 1), lambda b: (0, 0)),
            pl.BlockSpec((HIDDEN, HIDDEN), lambda b: (0, 0)),
            pl.BlockSpec((HIDDEN, 1), lambda b: (0, 0)),
        ],
        out_specs=[
            pl.BlockSpec((1, HIDDEN, S), lambda b: (b, 0, 0)),
            pl.BlockSpec((1, D_PC, S), lambda b: (b, 0, 0)),
            pl.BlockSpec((1, D_PC + HIDDEN, S), lambda b: (b, 0, 0)),
        ],
        out_shape=[
            jax.ShapeDtypeStruct((B, HIDDEN, S), jnp.float32),
            jax.ShapeDtypeStruct((B, D_PC, S), jnp.float32),
            jax.ShapeDtypeStruct((B, D_PC + HIDDEN, S), jnp.float32),
        ],
    )(feat, W1, b1c, W2, b2c)


def kernel(seed_xyz, pc_features, seed_inds, img_feats, vote_mask,
           W1, b1, W2, b2):
    sample, nvalid = _select(vote_mask)
    feat, xyz_flat, inds_flat = _gather(
        sample, nvalid, pc_features, img_feats,
        seed_xyz.reshape(B * N * 3), seed_inds.reshape(B * N))
    xyz = xyz_flat.reshape(B, S, 3)
    inds = inds_flat.reshape(B, S)
    img_out, pc_f, joint = _mlp(feat, W1, b1.reshape(HIDDEN, 1), W2,
                                b2.reshape(HIDDEN, 1))
    return (inds, xyz, feat, img_out, pc_f, joint)


# speedup vs baseline: 1.0499x; 1.0499x over previous
import jax
import jax.numpy as jnp
from jax import lax
from jax.experimental import pallas as pl
from jax.experimental.pallas import tpu as pltpu
from jax.experimental.pallas import tpu_sc as plsc

B = 4
N = 4096
K = 3
NK = N * K
D_PC = 256
D_IMG = 256
C_FEAT = D_PC + D_IMG + 1
C_PAD = 520
HIDDEN = 1024
S = 1024
L = 16
NVP = 128
NC = 2
NCHUNK = NK // L
CPR = N // L


def _select_body(vm_hbm, out_hbm, nv_hbm, mbuf, obuf, sbuf, nvbuf):
    wid = lax.axis_index("s") * NC + lax.axis_index("c")

    @pl.when(wid < B)
    def _():
        b = wid
        pltpu.sync_copy(vm_hbm.at[b], mbuf)
        lanes = lax.iota(jnp.int32, L)

        def body(i, carry):
            c1, c0 = carry
            kr = i // CPR
            col = (i - kr * CPR) * L
            v = mbuf[kr, pl.ds(col, L)]
            m = v != 0.0
            mi = m.astype(jnp.int32)
            cs = plsc.cumsum(mi)
            ii = lanes + 1
            pos_valid = c1 + cs - 1
            pos_invalid = (NK - 1) - (c0 + (ii - cs) - 1)
            pos = jnp.where(m, pos_valid, pos_invalid)
            plsc.store_scatter(obuf, [pos], i * L + lanes)
            nv = jnp.sum(mi)
            return c1 + nv, c0 + (L - nv)

        c1, _ = lax.fori_loop(0, NCHUNK, body, (jnp.int32(0), jnp.int32(0)))

        def fin(j, carry):
            p = j * L + lanes
            in_valid = p < c1
            src = jnp.where(in_valid, p, (NK - 1) - (p - c1))
            sbuf[pl.ds(j * L, L)] = plsc.load_gather(obuf, [src])
            return carry

        lax.fori_loop(0, S // L, fin, 0)
        pltpu.sync_copy(sbuf, out_hbm.at[pl.ds(b * S, S)])
        nvbuf[pl.ds(0, L)] = jnp.full((L,), 0, jnp.int32) + c1
        pltpu.sync_copy(nvbuf, nv_hbm.at[pl.ds(b * NVP, NVP)])


_select = pl.kernel(
    _select_body,
    out_type=(
        jax.ShapeDtypeStruct((B * S,), jnp.int32),
        jax.ShapeDtypeStruct((B * NVP,), jnp.int32),
    ),
    mesh=plsc.VectorSubcoreMesh(core_axis_name="c", subcore_axis_name="s"),
    compiler_params=pltpu.CompilerParams(
        needs_layout_passes=False, use_tc_tiling_on_sc=True),
    scratch_types=[
        pltpu.VMEM((K, N), jnp.float32),
        pltpu.VMEM((NK,), jnp.int32),
        pltpu.VMEM((S,), jnp.int32),
        pltpu.VMEM((NVP,), jnp.int32),
    ],
)


def _gather_body(sample, nv_in, pc, img, sxyz, sinds,
                 feat_out, xyz_out, inds_out,
                 n_buf, k_buf, stage2, out_buf, ibuf, xbuf, xout, nvbuf,
                 sem_a, sem_b, sem_o):
    wid = lax.axis_index("s") * NC + lax.axis_index("c")
    b = wid >> 3
    t = wid & 7
    lanes = lax.iota(jnp.int32, L)

    pltpu.sync_copy(sample.at[pl.ds(b * S, S)], n_buf)

    def split(j, carry):
        s = n_buf[pl.ds(j * L, L)]
        n_buf[pl.ds(j * L, L)] = s & (N - 1)
        k_buf[pl.ds(j * L, L)] = s >> 12
        return carry

    lax.fori_loop(0, S // L, split, 0)

    c_base = t * 32

    blocks = [("pc", i) for i in range(8)] + [("img", i) for i in range(8)]
    sems = (sem_a, sem_b)

    def start_stage(idx):
        kind, cb = blocks[idx]
        sl = idx % 2
        c0 = c_base + 4 * cb
        if kind == "pc":
            return [pltpu.async_copy(pc.at[b, pl.ds(c0, 4), :],
                                     stage2.at[sl, 0], sems[sl])]
        return [pltpu.async_copy(img.at[b, k, pl.ds(c0, 4), :],
                                 stage2.at[sl, k], sems[sl])
                for k in range(K)]

    pending = start_stage(0)
    out_pending = []
    for idx in range(16):
        sl = idx % 2
        nxt = start_stage(idx + 1) if idx + 1 < 16 else []
        for d in pending:
            d.wait()
        pending = nxt
        if sl == 0:
            for d in out_pending:
                d.wait()
            out_pending = []
        kind, cb = blocks[idx]
        if kind == "pc":
            def g(j, carry, sl=sl):
                nv = n_buf[pl.ds(j * L, L)]
                for c in range(4):
                    out_buf[sl * 4 + c, pl.ds(j * L, L)] = plsc.load_gather(
                        stage2.at[sl, 0], [jnp.full((L,), c, jnp.int32), nv])
                return carry
        else:
            def g(j, carry, sl=sl):
                nv = n_buf[pl.ds(j * L, L)]
                kv = k_buf[pl.ds(j * L, L)]
                for c in range(4):
                    out_buf[sl * 4 + c, pl.ds(j * L, L)] = plsc.load_gather(
                        stage2.at[sl], [kv, jnp.full((L,), c, jnp.int32), nv])
                return carry
        lax.fori_loop(0, S // L, g, 0)
        if sl == 1:
            kind0, cb0 = blocks[idx - 1]
            row0 = c_base + 4 * cb0 + (0 if kind0 == "pc" else D_PC)
            out_pending = [pltpu.async_copy(
                out_buf, feat_out.at[b, pl.ds(row0, 8), :], sem_o)]
    for d in out_pending:
        d.wait()

    @pl.when(t == 0)
    def _mask_row():
        pltpu.sync_copy(nv_in.at[pl.ds(b * NVP, NVP)], nvbuf)
        nvv = nvbuf[pl.ds(0, L)]

        def gm(j, carry):
            p = j * L + lanes
            out_buf[0, pl.ds(j * L, L)] = (p < nvv).astype(jnp.float32)
            return carry

        lax.fori_loop(0, S // L, gm, 0)
        pltpu.sync_copy(out_buf, feat_out.at[b, pl.ds(C_FEAT - 1, 8), :])

    @pl.when(t == 1)
    def _inds_row():
        pltpu.sync_copy(sinds.at[pl.ds(b * N, N)], ibuf)

        def gi(j, carry):
            nv = n_buf[pl.ds(j * L, L)]
            k_buf[pl.ds(j * L, L)] = plsc.load_gather(ibuf, [nv])
            return carry

        lax.fori_loop(0, S // L, gi, 0)
        pltpu.sync_copy(k_buf, inds_out.at[pl.ds(b * S, S)])

    @pl.when(t == 2)
    def _xyz_rows():
        pltpu.sync_copy(sxyz.at[pl.ds(b * N * 3, N * 3)], xbuf)

        def gx(j, carry):
            p = j * L + lanes
            jv = p // 3
            dv = p - jv * 3
            nv = plsc.load_gather(n_buf, [jv])
            xout[pl.ds(j * L, L)] = plsc.load_gather(xbuf, [nv * 3 + dv])
            return carry

        lax.fori_loop(0, (S * 3) // L, gx, 0)
        pltpu.sync_copy(xout, xyz_out.at[pl.ds(b * S * 3, S * 3)])


_gather = pl.kernel(
    _gather_body,
    out_type=(
        jax.ShapeDtypeStruct((B, C_PAD, S), jnp.float32),
        jax.ShapeDtypeStruct((B * S * 3,), jnp.float32),
        jax.ShapeDtypeStruct((B * S,), jnp.int32),
    ),
    mesh=plsc.VectorSubcoreMesh(core_axis_name="c", subcore_axis_name="s"),
    compiler_params=pltpu.CompilerParams(
        needs_layout_passes=False, use_tc_tiling_on_sc=True),
    scratch_types=[
        pltpu.VMEM((S,), jnp.int32),
        pltpu.VMEM((S,), jnp.int32),
        pltpu.VMEM((2, K, 4, N), jnp.float32),
        pltpu.VMEM((8, S), jnp.float32),
        pltpu.VMEM((N,), jnp.int32),
        pltpu.VMEM((N * 3,), jnp.float32),
        pltpu.VMEM((S * 3,), jnp.float32),
        pltpu.VMEM((NVP,), jnp.int32),
        pltpu.SemaphoreType.DMA,
        pltpu.SemaphoreType.DMA,
        pltpu.SemaphoreType.DMA,
    ],
)


def _mlp_body(feat_ref, w1_ref, b1_ref, w2_ref, b2_ref,
              img_out_ref, pc_ref, joint_ref):
    x = feat_ref[0, D_PC:C_FEAT, :]
    h = jnp.dot(w1_ref[...].astype(jnp.bfloat16), x.astype(jnp.bfloat16),
                preferred_element_type=jnp.float32)
    h = jnp.maximum(h + b1_ref[...], 0.0)
    o = jnp.dot(w2_ref[...].astype(jnp.bfloat16), h.astype(jnp.bfloat16),
                preferred_element_type=jnp.float32)
    o = jnp.maximum(o + b2_ref[...], 0.0)
    img_out_ref[0] = o
    pcv = feat_ref[0, :D_PC, :]
    pc_ref[0] = pcv
    joint_ref[0, :D_PC, :] = pcv
    joint_ref[0, D_PC:, :] = o


def _mlp(feat, W1, b1c, W2, b2c):
    return pl.pallas_call(
        _mlp_body,
        grid=(B,),
        in_specs=[
            pl.BlockSpec((1, C_PAD, S), lambda b: (b, 0, 0)),
            pl.BlockSpec((HIDDEN, D_IMG + 1), lambda b: (0, 0)),
            pl.BlockSpec((HIDDEN, 1), lambda b: (0, 0)),
            pl.BlockSpec((HIDDEN, HIDDEN), lambda b: (0, 0)),
            pl.BlockSpec((HIDDEN, 1), lambda b: (0, 0)),
        ],
        out_specs=[
            pl.BlockSpec((1, HIDDEN, S), lambda b: (b, 0, 0)),
            pl.BlockSpec((1, D_PC, S), lambda b: (b, 0, 0)),
            pl.BlockSpec((1, D_PC + HIDDEN, S), lambda b: (b, 0, 0)),
        ],
        out_shape=[
            jax.ShapeDtypeStruct((B, HIDDEN, S), jnp.float32),
            jax.ShapeDtypeStruct((B, D_PC, S), jnp.float32),
            jax.ShapeDtypeStruct((B, D_PC + HIDDEN, S), jnp.float32),
        ],
    )(feat, W1, b1c, W2, b2c)


def kernel(seed_xyz, pc_features, seed_inds, img_feats, vote_mask,
           W1, b1, W2, b2):
    sample, nvalid = _select(vote_mask)
    feat_pad, xyz_flat, inds_flat = _gather(
        sample, nvalid, pc_features, img_feats,
        seed_xyz.reshape(B * N * 3), seed_inds.reshape(B * N))
    feat = lax.slice(feat_pad, (0, 0, 0), (B, C_FEAT, S))
    xyz = xyz_flat.reshape(B, S, 3)
    inds = inds_flat.reshape(B, S)
    img_out, pc_f, joint = _mlp(feat_pad, W1, b1.reshape(HIDDEN, 1), W2,
                                b2.reshape(HIDDEN, 1))
    return (inds, xyz, feat, img_out, pc_f, joint)


# speedup vs baseline: 1.0557x; 1.0055x over previous
import jax
import jax.numpy as jnp
from jax import lax
from jax.experimental import pallas as pl
from jax.experimental.pallas import tpu as pltpu
from jax.experimental.pallas import tpu_sc as plsc

B = 4
N = 4096
K = 3
NK = N * K
D_PC = 256
D_IMG = 256
C_FEAT = D_PC + D_IMG + 1
C_PAD = 520
HIDDEN = 1024
S = 1024
L = 16
NVP = 128
NC = 2
NCHUNK = NK // L
CPR = N // L


def _select_body(vm_hbm, out_hbm, nv_hbm, mbuf, obuf, sbuf, nvbuf):
    wid = lax.axis_index("s") * NC + lax.axis_index("c")

    @pl.when(wid < B)
    def _():
        b = wid
        pltpu.sync_copy(vm_hbm.at[b], mbuf)
        lanes = lax.iota(jnp.int32, L)

        def body(i, carry):
            c1, c0 = carry
            kr = i // CPR
            col = (i - kr * CPR) * L
            v = mbuf[kr, pl.ds(col, L)]
            m = v != 0.0
            mi = m.astype(jnp.int32)
            cs = plsc.cumsum(mi)
            ii = lanes + 1
            pos_valid = c1 + cs - 1
            pos_invalid = (NK - 1) - (c0 + (ii - cs) - 1)
            pos = jnp.where(m, pos_valid, pos_invalid)
            plsc.store_scatter(obuf, [pos], i * L + lanes)
            nv = jnp.sum(mi)
            return c1 + nv, c0 + (L - nv)

        c1, _ = lax.fori_loop(0, NCHUNK, body, (jnp.int32(0), jnp.int32(0)))

        def fin(j, carry):
            p = j * L + lanes
            in_valid = p < c1
            src = jnp.where(in_valid, p, (NK - 1) - (p - c1))
            sbuf[pl.ds(j * L, L)] = plsc.load_gather(obuf, [src])
            return carry

        lax.fori_loop(0, S // L, fin, 0)
        pltpu.sync_copy(sbuf, out_hbm.at[pl.ds(b * S, S)])
        nvbuf[pl.ds(0, L)] = jnp.full((L,), 0, jnp.int32) + c1
        pltpu.sync_copy(nvbuf, nv_hbm.at[pl.ds(b * NVP, NVP)])


_select = pl.kernel(
    _select_body,
    out_type=(
        jax.ShapeDtypeStruct((B * S,), jnp.int32),
        jax.ShapeDtypeStruct((B * NVP,), jnp.int32),
    ),
    mesh=plsc.VectorSubcoreMesh(core_axis_name="c", subcore_axis_name="s"),
    compiler_params=pltpu.CompilerParams(
        needs_layout_passes=False, use_tc_tiling_on_sc=False),
    scratch_types=[
        pltpu.VMEM((K, N), jnp.float32),
        pltpu.VMEM((NK,), jnp.int32),
        pltpu.VMEM((S,), jnp.int32),
        pltpu.VMEM((NVP,), jnp.int32),
    ],
)


def _gather_body(sample, nv_in, pc, img, sxyz, sinds,
                 feat_out, xyz_out, inds_out,
                 n_buf, k_buf, stage2, out_buf, ibuf, xbuf, xout, nvbuf,
                 sem_a, sem_b, sem_o):
    wid = lax.axis_index("s") * NC + lax.axis_index("c")
    b = wid >> 3
    t = wid & 7
    lanes = lax.iota(jnp.int32, L)

    pltpu.sync_copy(sample.at[pl.ds(b * S, S)], n_buf)

    def split(j, carry):
        s = n_buf[pl.ds(j * L, L)]
        n_buf[pl.ds(j * L, L)] = s & (N - 1)
        k_buf[pl.ds(j * L, L)] = s >> 12
        return carry

    lax.fori_loop(0, S // L, split, 0)

    c_base = t * 32

    blocks = [("pc", i) for i in range(8)] + [("img", i) for i in range(8)]
    sems = (sem_a, sem_b)

    def start_stage(idx):
        kind, cb = blocks[idx]
        sl = idx % 2
        c0 = c_base + 4 * cb
        if kind == "pc":
            return [pltpu.async_copy(pc.at[b, pl.ds(c0, 4), :],
                                     stage2.at[sl, 0], sems[sl])]
        return [pltpu.async_copy(img.at[b, k, pl.ds(c0, 4), :],
                                 stage2.at[sl, k], sems[sl])
                for k in range(K)]

    pending = start_stage(0)
    out_pending = []
    for idx in range(16):
        sl = idx % 2
        nxt = start_stage(idx + 1) if idx + 1 < 16 else []
        for d in pending:
            d.wait()
        pending = nxt
        if sl == 0:
            for d in out_pending:
                d.wait()
            out_pending = []
        kind, cb = blocks[idx]
        if kind == "pc":
            def g(j, carry, sl=sl):
                nv = n_buf[pl.ds(j * L, L)]
                for c in range(4):
                    out_buf[sl * 4 + c, pl.ds(j * L, L)] = plsc.load_gather(
                        stage2.at[sl, 0], [jnp.full((L,), c, jnp.int32), nv])
                return carry
        else:
            def g(j, carry, sl=sl):
                nv = n_buf[pl.ds(j * L, L)]
                kv = k_buf[pl.ds(j * L, L)]
                for c in range(4):
                    out_buf[sl * 4 + c, pl.ds(j * L, L)] = plsc.load_gather(
                        stage2.at[sl], [kv, jnp.full((L,), c, jnp.int32), nv])
                return carry
        lax.fori_loop(0, S // L, g, 0)
        if sl == 1:
            kind0, cb0 = blocks[idx - 1]
            row0 = c_base + 4 * cb0 + (0 if kind0 == "pc" else D_PC)
            out_pending = [pltpu.async_copy(
                out_buf, feat_out.at[b, pl.ds(row0, 8), :], sem_o)]
    for d in out_pending:
        d.wait()

    @pl.when(t == 0)
    def _mask_row():
        pltpu.sync_copy(nv_in.at[pl.ds(b * NVP, NVP)], nvbuf)
        nvv = nvbuf[pl.ds(0, L)]

        def gm(j, carry):
            p = j * L + lanes
            out_buf[0, pl.ds(j * L, L)] = (p < nvv).astype(jnp.float32)
            return carry

        lax.fori_loop(0, S // L, gm, 0)
        pltpu.sync_copy(out_buf, feat_out.at[b, pl.ds(C_FEAT - 1, 8), :])

    @pl.when(t == 1)
    def _inds_row():
        pltpu.sync_copy(sinds.at[pl.ds(b * N, N)], ibuf)

        def gi(j, carry):
            nv = n_buf[pl.ds(j * L, L)]
            k_buf[pl.ds(j * L, L)] = plsc.load_gather(ibuf, [nv])
            return carry

        lax.fori_loop(0, S // L, gi, 0)
        pltpu.sync_copy(k_buf, inds_out.at[pl.ds(b * S, S)])

    @pl.when(t == 2)
    def _xyz_rows():
        pltpu.sync_copy(sxyz.at[pl.ds(b * N * 3, N * 3)], xbuf)

        def gx(j, carry):
            p = j * L + lanes
            jv = p // 3
            dv = p - jv * 3
            nv = plsc.load_gather(n_buf, [jv])
            xout[pl.ds(j * L, L)] = plsc.load_gather(xbuf, [nv * 3 + dv])
            return carry

        lax.fori_loop(0, (S * 3) // L, gx, 0)
        pltpu.sync_copy(xout, xyz_out.at[pl.ds(b * S * 3, S * 3)])


_gather = pl.kernel(
    _gather_body,
    out_type=(
        jax.ShapeDtypeStruct((B, C_PAD, S), jnp.float32),
        jax.ShapeDtypeStruct((B * S * 3,), jnp.float32),
        jax.ShapeDtypeStruct((B * S,), jnp.int32),
    ),
    mesh=plsc.VectorSubcoreMesh(core_axis_name="c", subcore_axis_name="s"),
    compiler_params=pltpu.CompilerParams(
        needs_layout_passes=False, use_tc_tiling_on_sc=True),
    scratch_types=[
        pltpu.VMEM((S,), jnp.int32),
        pltpu.VMEM((S,), jnp.int32),
        pltpu.VMEM((2, K, 4, N), jnp.float32),
        pltpu.VMEM((8, S), jnp.float32),
        pltpu.VMEM((N,), jnp.int32),
        pltpu.VMEM((N * 3,), jnp.float32),
        pltpu.VMEM((S * 3,), jnp.float32),
        pltpu.VMEM((NVP,), jnp.int32),
        pltpu.SemaphoreType.DMA,
        pltpu.SemaphoreType.DMA,
        pltpu.SemaphoreType.DMA,
    ],
)


def _mlp_body(feat_ref, w1_ref, b1_ref, w2_ref, b2_ref,
              img_out_ref, pc_ref, joint_ref):
    x = feat_ref[0, D_PC:C_FEAT, :]
    h = jnp.dot(w1_ref[...].astype(jnp.bfloat16), x.astype(jnp.bfloat16),
                preferred_element_type=jnp.float32)
    h = jnp.maximum(h + b1_ref[...], 0.0)
    o = jnp.dot(w2_ref[...].astype(jnp.bfloat16), h.astype(jnp.bfloat16),
                preferred_element_type=jnp.float32)
    o = jnp.maximum(o + b2_ref[...], 0.0)
    img_out_ref[0] = o
    pcv = feat_ref[0, :D_PC, :]
    pc_ref[0] = pcv
    joint_ref[0, :D_PC, :] = pcv
    joint_ref[0, D_PC:, :] = o


def _mlp(feat, W1, b1c, W2, b2c):
    return pl.pallas_call(
        _mlp_body,
        grid=(B,),
        in_specs=[
            pl.BlockSpec((1, C_PAD, S), lambda b: (b, 0, 0)),
            pl.BlockSpec((HIDDEN, D_IMG + 1), lambda b: (0, 0)),
            pl.BlockSpec((HIDDEN, 1), lambda b: (0, 0)),
            pl.BlockSpec((HIDDEN, HIDDEN), lambda b: (0, 0)),
            pl.BlockSpec((HIDDEN, 1), lambda b: (0, 0)),
        ],
        out_specs=[
            pl.BlockSpec((1, HIDDEN, S), lambda b: (b, 0, 0)),
            pl.BlockSpec((1, D_PC, S), lambda b: (b, 0, 0)),
            pl.BlockSpec((1, D_PC + HIDDEN, S), lambda b: (b, 0, 0)),
        ],
        out_shape=[
            jax.ShapeDtypeStruct((B, HIDDEN, S), jnp.float32),
            jax.ShapeDtypeStruct((B, D_PC, S), jnp.float32),
            jax.ShapeDtypeStruct((B, D_PC + HIDDEN, S), jnp.float32),
        ],
    )(feat, W1, b1c, W2, b2c)


def kernel(seed_xyz, pc_features, seed_inds, img_feats, vote_mask,
           W1, b1, W2, b2):
    sample, nvalid = _select(vote_mask)
    feat_pad, xyz_flat, inds_flat = _gather(
        sample, nvalid, pc_features, img_feats,
        seed_xyz.reshape(B * N * 3), seed_inds.reshape(B * N))
    feat = lax.slice(feat_pad, (0, 0, 0), (B, C_FEAT, S))
    xyz = xyz_flat.reshape(B, S, 3)
    inds = inds_flat.reshape(B, S)
    img_out, pc_f, joint = _mlp(feat_pad, W1, b1.reshape(HIDDEN, 1), W2,
                                b2.reshape(HIDDEN, 1))
    return (inds, xyz, feat, img_out, pc_f, joint)
